# initial kernel scaffold (unmeasured)
import jax
import jax.numpy as jnp
from jax import lax
from jax.experimental import pallas as pl
from jax.experimental.pallas import tpu as pltpu

N_DEV = 8
SQ = 256
DM = 1024
HQL = 8
DH = 128
NG = 16
RES = 4
KB = 64
SCALE = 0.08838834764831843

_BITS = (4, 2, 1)


def kernel(x, Wq, K_ext, V_ext, Wo):
    x2 = x.reshape(SQ, DM)
    K2 = K_ext.reshape(NG, RES, KB, 64, DH)
    V2 = V_ext.reshape(NG, RES, KB, 64, DH)

    def body(x_ref, wq_ref, k_hbm, v_hbm, wo_ref, out_ref,
             kbuf, vbuf, ctx_buf, rbuf,
             ksems, vsems, send_sems, recv_sems):
        my = lax.axis_index("i")
        h0 = my * HQL

        kcopies = {}
        vcopies = {}
        for qb in range(RES):
            for h in range(HQL):
                hg = h0 + h
                kc = pltpu.make_async_copy(
                    k_hbm.at[:, qb, :, hg, :], kbuf.at[qb, h], ksems.at[qb, h]
                )
                vc = pltpu.make_async_copy(
                    v_hbm.at[:, qb, :, hg, :], vbuf.at[qb, h], vsems.at[qb, h]
                )
                kc.start()
                vc.start()
                kcopies[qb, h] = kc
                vcopies[qb, h] = vc

        q = jnp.dot(x_ref[:, :], wq_ref[:, :],
                    preferred_element_type=jnp.float32) * SCALE

        for qb in range(RES):
            for h in range(HQL):
                kcopies[qb, h].wait()
                vcopies[qb, h].wait()
                kv_k = kbuf[qb, h].reshape(NG * KB, DH)
                kv_v = vbuf[qb, h].reshape(NG * KB, DH)
                qh = q[qb * KB:(qb + 1) * KB, h * DH:(h + 1) * DH]
                s = lax.dot_general(
                    qh, kv_k, (((1,), (1,)), ((), ())),
                    preferred_element_type=jnp.float32,
                )
                m = jnp.max(s, axis=1, keepdims=True)
                e = jnp.exp(s - m)
                p = e / jnp.sum(e, axis=1, keepdims=True)
                ctx = jnp.dot(p, kv_v, preferred_element_type=jnp.float32)
                ctx_buf[qb * KB:(qb + 1) * KB, h * DH:(h + 1) * DH] = ctx

        out_ref[0] = jnp.dot(ctx_buf[:, :], wo_ref[:, :],
                             preferred_element_type=jnp.float32)

        barrier_sem = pltpu.get_barrier_semaphore()
        for b in _BITS:
            pl.semaphore_signal(
                barrier_sem, inc=1,
                device_id=(my ^ b,), device_id_type=pl.DeviceIdType.MESH,
            )
        pl.semaphore_wait(barrier_sem, len(_BITS))

        for r, b in enumerate(_BITS):
            partner = my ^ b
            rdma = pltpu.make_async_remote_copy(
                src_ref=out_ref.at[0],
                dst_ref=rbuf.at[r],
                send_sem=send_sems.at[r],
                recv_sem=recv_sems.at[r],
                device_id=(partner,),
                device_id_type=pl.DeviceIdType.MESH,
            )
            rdma.start()
            rdma.wait()
            out_ref[0] += rbuf[r]

    return pl.pallas_call(
        body,
        out_shape=jax.ShapeDtypeStruct((1, SQ, DM), jnp.float32),
        in_specs=[
            pl.BlockSpec(memory_space=pltpu.VMEM),
            pl.BlockSpec(memory_space=pltpu.VMEM),
            pl.BlockSpec(memory_space=pltpu.ANY),
            pl.BlockSpec(memory_space=pltpu.ANY),
            pl.BlockSpec(memory_space=pltpu.VMEM),
        ],
        out_specs=pl.BlockSpec(memory_space=pltpu.VMEM),
        scratch_shapes=[
            pltpu.VMEM((RES, HQL, NG, KB, DH), jnp.float32),
            pltpu.VMEM((RES, HQL, NG, KB, DH), jnp.float32),
            pltpu.VMEM((SQ, DM), jnp.float32),
            pltpu.VMEM((len(_BITS), SQ, DM), jnp.float32),
            pltpu.SemaphoreType.DMA((RES, HQL)),
            pltpu.SemaphoreType.DMA((RES, HQL)),
            pltpu.SemaphoreType.DMA((len(_BITS),)),
            pltpu.SemaphoreType.DMA((len(_BITS),)),
        ],
        compiler_params=pltpu.CompilerParams(
            collective_id=0,
            vmem_limit_bytes=110 * 1024 * 1024,
        ),
    )(x2, Wq, K2, V2, Wo)


# baseline (device time: 71375 ns/iter reference)
import jax
import jax.numpy as jnp
from jax import lax
from jax.experimental import pallas as pl
from jax.experimental.pallas import tpu as pltpu

N_DEV = 8
SQ = 256
DM = 1024
HQL = 8
DH = 128
NG = 16
RES = 4
KB = 64
SCALE = 0.08838834764831843

_BITS = (4, 2, 1)


def kernel(x, Wq, K_ext, V_ext, Wo):
    x2 = x.reshape(SQ, DM)
    K2 = K_ext.reshape(NG, RES, KB, 64, DH)
    V2 = V_ext.reshape(NG, RES, KB, 64, DH)

    def body(x_ref, wq_ref, k_hbm, v_hbm, wo_ref, out_ref,
             kbuf, vbuf, ctx_buf, rbuf,
             ksems, vsems, send_sems, recv_sems):
        my = lax.axis_index("i")
        h0 = my * HQL

        kcopies = {}
        vcopies = {}
        for qb in range(RES):
            for h in range(HQL):
                hg = h0 + h
                kc = pltpu.make_async_copy(
                    k_hbm.at[:, qb, :, hg, :], kbuf.at[qb, h], ksems.at[qb, h]
                )
                vc = pltpu.make_async_copy(
                    v_hbm.at[:, qb, :, hg, :], vbuf.at[qb, h], vsems.at[qb, h]
                )
                kc.start()
                vc.start()
                kcopies[qb, h] = kc
                vcopies[qb, h] = vc

        q = jnp.dot(x_ref[:, :], wq_ref[:, :],
                    preferred_element_type=jnp.float32) * SCALE

        for qb in range(RES):
            for h in range(HQL):
                kcopies[qb, h].wait()
                vcopies[qb, h].wait()
                kv_k = kbuf[qb, h].reshape(NG * KB, DH)
                kv_v = vbuf[qb, h].reshape(NG * KB, DH)
                qh = q[qb * KB:(qb + 1) * KB, h * DH:(h + 1) * DH]
                s = lax.dot_general(
                    qh, kv_k, (((1,), (1,)), ((), ())),
                    preferred_element_type=jnp.float32,
                )
                m = jnp.max(s, axis=1, keepdims=True)
                e = jnp.exp(s - m)
                p = e / jnp.sum(e, axis=1, keepdims=True)
                ctx = jnp.dot(p, kv_v, preferred_element_type=jnp.float32)
                ctx_buf[qb * KB:(qb + 1) * KB, h * DH:(h + 1) * DH] = ctx

        out_ref[0] = jnp.dot(ctx_buf[:, :], wo_ref[:, :],
                             preferred_element_type=jnp.float32)

        barrier_sem = pltpu.get_barrier_semaphore()
        for b in _BITS:
            pl.semaphore_signal(
                barrier_sem, inc=1,
                device_id=(my ^ b,), device_id_type=pl.DeviceIdType.MESH,
            )
        pl.semaphore_wait(barrier_sem, len(_BITS))

        for r, b in enumerate(_BITS):
            partner = my ^ b
            rdma = pltpu.make_async_remote_copy(
                src_ref=out_ref.at[0],
                dst_ref=rbuf.at[r],
                send_sem=send_sems.at[r],
                recv_sem=recv_sems.at[r],
                device_id=(partner,),
                device_id_type=pl.DeviceIdType.MESH,
            )
            rdma.start()
            rdma.wait()
            out_ref[0] += rbuf[r]

    return pl.pallas_call(
        body,
        out_shape=jax.ShapeDtypeStruct((1, SQ, DM), jnp.float32),
        in_specs=[
            pl.BlockSpec(memory_space=pltpu.VMEM),
            pl.BlockSpec(memory_space=pltpu.VMEM),
            pl.BlockSpec(memory_space=pl.ANY),
            pl.BlockSpec(memory_space=pl.ANY),
            pl.BlockSpec(memory_space=pltpu.VMEM),
        ],
        out_specs=pl.BlockSpec(memory_space=pltpu.VMEM),
        scratch_shapes=[
            pltpu.VMEM((RES, HQL, NG, KB, DH), jnp.float32),
            pltpu.VMEM((RES, HQL, NG, KB, DH), jnp.float32),
            pltpu.VMEM((SQ, DM), jnp.float32),
            pltpu.VMEM((len(_BITS), SQ, DM), jnp.float32),
            pltpu.SemaphoreType.DMA((RES, HQL)),
            pltpu.SemaphoreType.DMA((RES, HQL)),
            pltpu.SemaphoreType.DMA((len(_BITS),)),
            pltpu.SemaphoreType.DMA((len(_BITS),)),
        ],
        compiler_params=pltpu.CompilerParams(
            collective_id=0,
            vmem_limit_bytes=110 * 1024 * 1024,
        ),
    )(x2, Wq, K2, V2, Wo)


# device time: 25612 ns/iter; 2.7868x vs baseline; 2.7868x over previous
import jax
import jax.numpy as jnp
from jax import lax
from jax.experimental import pallas as pl
from jax.experimental.pallas import tpu as pltpu

N_DEV = 8
SQ = 256
DM = 1024
HQL = 8
DH = 128
NG = 16
RES = 4
KB = 64
SCALE = 0.08838834764831843

_BITS = (4, 2, 1)

import os
_SKIP_AR = bool(int(os.environ.get("SKIP_AR", "0")))


def kernel(x, Wq, K_ext, V_ext, Wo):
    x2 = x.reshape(SQ, DM)
    K2 = K_ext.reshape(NG, RES, KB, 64, DH)
    V2 = V_ext.reshape(NG, RES, KB, 64, DH)

    def body(x_ref, wq_ref, k_hbm, v_hbm, wo_ref, out_ref,
             kbuf, vbuf, ctx_buf, rbuf,
             ksems, vsems, send_sems, recv_sems):
        my = lax.axis_index("i")
        h0 = my * HQL

        kcopies = {}
        vcopies = {}
        for qb in range(RES):
            for h in range(HQL):
                hg = h0 + h
                kc = pltpu.make_async_copy(
                    k_hbm.at[:, qb, :, hg, :], kbuf.at[qb, h], ksems.at[qb, h]
                )
                vc = pltpu.make_async_copy(
                    v_hbm.at[:, qb, :, hg, :], vbuf.at[qb, h], vsems.at[qb, h]
                )
                kc.start()
                vc.start()
                kcopies[qb, h] = kc
                vcopies[qb, h] = vc

        q = jnp.dot(x_ref[:, :], wq_ref[:, :],
                    preferred_element_type=jnp.float32) * SCALE

        for qb in range(RES):
            for h in range(HQL):
                kcopies[qb, h].wait()
                vcopies[qb, h].wait()
                kv_k = kbuf[qb, h].reshape(NG * KB, DH)
                kv_v = vbuf[qb, h].reshape(NG * KB, DH)
                qh = q[qb * KB:(qb + 1) * KB, h * DH:(h + 1) * DH]
                s = lax.dot_general(
                    qh, kv_k, (((1,), (1,)), ((), ())),
                    preferred_element_type=jnp.float32,
                )
                m = jnp.max(s, axis=1, keepdims=True)
                e = jnp.exp(s - m)
                p = e / jnp.sum(e, axis=1, keepdims=True)
                ctx = jnp.dot(p, kv_v, preferred_element_type=jnp.float32)
                ctx_buf[qb * KB:(qb + 1) * KB, h * DH:(h + 1) * DH] = ctx

        out_ref[0] = jnp.dot(ctx_buf[:, :], wo_ref[:, :],
                             preferred_element_type=jnp.float32)

        if _SKIP_AR:
            return
        barrier_sem = pltpu.get_barrier_semaphore()
        for b in _BITS:
            pl.semaphore_signal(
                barrier_sem, inc=1,
                device_id=(my ^ b,), device_id_type=pl.DeviceIdType.MESH,
            )
        pl.semaphore_wait(barrier_sem, len(_BITS))

        for r, b in enumerate(_BITS):
            partner = my ^ b
            rdma = pltpu.make_async_remote_copy(
                src_ref=out_ref.at[0],
                dst_ref=rbuf.at[r],
                send_sem=send_sems.at[r],
                recv_sem=recv_sems.at[r],
                device_id=(partner,),
                device_id_type=pl.DeviceIdType.MESH,
            )
            rdma.start()
            rdma.wait()
            out_ref[0] += rbuf[r]

    return pl.pallas_call(
        body,
        out_shape=jax.ShapeDtypeStruct((1, SQ, DM), jnp.float32),
        in_specs=[
            pl.BlockSpec(memory_space=pltpu.VMEM),
            pl.BlockSpec(memory_space=pltpu.VMEM),
            pl.BlockSpec(memory_space=pl.ANY),
            pl.BlockSpec(memory_space=pl.ANY),
            pl.BlockSpec(memory_space=pltpu.VMEM),
        ],
        out_specs=pl.BlockSpec(memory_space=pltpu.VMEM),
        scratch_shapes=[
            pltpu.VMEM((RES, HQL, NG, KB, DH), jnp.float32),
            pltpu.VMEM((RES, HQL, NG, KB, DH), jnp.float32),
            pltpu.VMEM((SQ, DM), jnp.float32),
            pltpu.VMEM((len(_BITS), SQ, DM), jnp.float32),
            pltpu.SemaphoreType.DMA((RES, HQL)),
            pltpu.SemaphoreType.DMA((RES, HQL)),
            pltpu.SemaphoreType.DMA((len(_BITS),)),
            pltpu.SemaphoreType.DMA((len(_BITS),)),
        ],
        compiler_params=pltpu.CompilerParams(
            collective_id=None if _SKIP_AR else 0,
            vmem_limit_bytes=110 * 1024 * 1024,
        ),
    )(x2, Wq, K2, V2, Wo)
